# baseline (speedup 1.0000x reference)
import functools

import jax
import jax.numpy as jnp
from jax import lax
from jax.experimental import pallas as pl
from jax.experimental.pallas import tpu as pltpu
from jax.experimental.pallas import tpu_sc as plsc

_B, _T, _N, _D = 8, 4, 1025, 768
_BT = _B * _T
_SLAB = _N * _D
_SROWS = 32
_STRIPE = _SROWS * _D
_TAIL_OFF = (_N - 1) * _D
_NVEC = _STRIPE // 16
_NVEC_TAIL = _D // 16


def _sc_body(x_hbm, ar_hbm, local_hbm, glob_hbm, gate_hbm, out_hbm,
             xb0, xb1, xb2, gbuf, lbuf, xtail, ltail, gtail, arv_b, gv_b,
             xsem, osem, gsem, tsem):
    xbufs = (xb0, xb1, xb2)
    wid = lax.axis_index("s") * 2 + lax.axis_index("c")
    stripe_off = wid * _STRIPE

    pltpu.sync_copy(ar_hbm, arv_b)
    pltpu.sync_copy(gate_hbm, gv_b)
    pltpu.sync_copy(local_hbm.at[pl.ds(stripe_off, _STRIPE)], lbuf)
    pltpu.sync_copy(local_hbm.at[pl.ds(_TAIL_OFF, _D)], ltail)

    gv = gv_b[...]
    arv = arv_b[...]
    tgv = 1.0 - 2.0 / (jnp.exp(2.0 * gv) + 1.0)
    lsv = 1.0 - tgv
    gate_nz = gv[0] != 0.0

    def _zero(i, _):
        gbuf[pl.ds(i * 16, 16)] = jnp.zeros((16,), jnp.float32)
        return 0
    lax.fori_loop(0, _NVEC, _zero, 0)

    def _zero_t(i, _):
        gtail[pl.ds(i * 16, 16)] = jnp.zeros((16,), jnp.float32)
        return 0
    lax.fori_loop(0, _NVEC_TAIL, _zero_t, 0)

    def slab_meta(b, t):
        h = arv[2 * b]
        w = arv[2 * b + 1]
        mask = t < h * w
        use = jnp.logical_and(mask, gate_nz)
        safe_w = jnp.maximum(w, 1)
        gidx = (t // safe_w) * 4 + t % safe_w
        coefv = tgv * jnp.full((16,), use.astype(jnp.float32))
        return use, coefv, gidx

    def x_in(s, slot):
        return pltpu.make_async_copy(
            x_hbm.at[pl.ds(s * _SLAB + stripe_off, _STRIPE)], xbufs[slot],
            xsem.at[slot])

    def x_out(s, slot):
        return pltpu.make_async_copy(
            xbufs[slot], out_hbm.at[pl.ds(s * _SLAB + stripe_off, _STRIPE)],
            osem.at[slot])

    x_in(0, 0).start()
    for s in range(_BT):
        slot = s % 3
        if s + 1 < _BT:
            nslot = (s + 1) % 3
            if s + 1 >= 3:
                x_out(s - 2, nslot).wait()
            x_in(s + 1, nslot).start()
        x_in(s, slot).wait()

        use, coefv, gidx = slab_meta(s // 4, s % 4)

        @pl.when(use)
        def _(gidx=gidx):
            pltpu.make_async_copy(
                glob_hbm.at[pl.ds(gidx * _SLAB + stripe_off, _STRIPE)], gbuf,
                gsem).start()
            pltpu.make_async_copy(
                glob_hbm.at[pl.ds(gidx * _SLAB + stripe_off, _STRIPE)], gbuf,
                gsem).wait()

        xb = xbufs[slot]

        def _step(i, _, xb=xb, coefv=coefv):
            sl = pl.ds(i * 16, 16)
            xb[sl] = xb[sl] + lbuf[sl] * lsv + gbuf[sl] * coefv
            return 0
        lax.fori_loop(0, _NVEC, _step, 0)
        x_out(s, slot).start()

        @pl.when(wid == s)
        def _(s=s, use=use, coefv=coefv, gidx=gidx):
            tail_base = s * _SLAB + _TAIL_OFF
            pltpu.sync_copy(x_hbm.at[pl.ds(tail_base, _D)], xtail)

            @pl.when(use)
            def _():
                pltpu.make_async_copy(
                    glob_hbm.at[pl.ds(gidx * _SLAB + _TAIL_OFF, _D)], gtail,
                    tsem).start()
                pltpu.make_async_copy(
                    glob_hbm.at[pl.ds(gidx * _SLAB + _TAIL_OFF, _D)], gtail,
                    tsem).wait()

            def _step_t(i, _):
                sl = pl.ds(i * 16, 16)
                xtail[sl] = xtail[sl] + ltail[sl] * lsv + gtail[sl] * coefv
                return 0
            lax.fori_loop(0, _NVEC_TAIL, _step_t, 0)
            pltpu.sync_copy(xtail, out_hbm.at[pl.ds(tail_base, _D)])

    for s in range(_BT - 3, _BT):
        x_out(s, s % 3).wait()


def _sc_call(x_flat, ar_flat, local_flat, glob_flat, gate16):
    mesh = plsc.VectorSubcoreMesh(core_axis_name="c", subcore_axis_name="s")
    f = pl.kernel(
        _sc_body,
        mesh=mesh,
        out_type=jax.ShapeDtypeStruct((_BT * _SLAB,), jnp.float32),
        scratch_types=[
            pltpu.VMEM((_STRIPE,), jnp.float32),
            pltpu.VMEM((_STRIPE,), jnp.float32),
            pltpu.VMEM((_STRIPE,), jnp.float32),
            pltpu.VMEM((_STRIPE,), jnp.float32),
            pltpu.VMEM((_STRIPE,), jnp.float32),
            pltpu.VMEM((_D,), jnp.float32),
            pltpu.VMEM((_D,), jnp.float32),
            pltpu.VMEM((_D,), jnp.float32),
            pltpu.VMEM((16,), jnp.int32),
            pltpu.VMEM((16,), jnp.float32),
            pltpu.SemaphoreType.DMA((3,)),
            pltpu.SemaphoreType.DMA((3,)),
            pltpu.SemaphoreType.DMA,
            pltpu.SemaphoreType.DMA,
        ],
    )
    return f(x_flat, ar_flat, local_flat, glob_flat, gate16)


def kernel(x, aspect_ratio, local_token_positional_embedding,
           global_token_positional_embedding, gate):
    B, T, N, D = x.shape
    x_flat = x.reshape(-1)
    ar_flat = jnp.broadcast_to(
        aspect_ratio.astype(jnp.int32).reshape(-1), (16,))
    local_flat = local_token_positional_embedding.reshape(-1)
    glob_flat = global_token_positional_embedding.reshape(-1)
    gate16 = jnp.broadcast_to(gate.astype(jnp.float32), (16,))
    out = _sc_call(x_flat, ar_flat, local_flat, glob_flat, gate16)
    return out.reshape(B, T, N, D)
